# Initial kernel scaffold; baseline (speedup 1.0000x reference)
import jax
import jax.numpy as jnp
from jax.experimental import pallas as pl


def kernel(x, edge_index, batch, W_pre, b_pre, W1a, b1a, W1b, b1b, W2a, b2a, W2b, b2b, W3a, b3a, W3b, b3b, Wp1, bp1, Wp2, bp2):
    raise NotImplementedError("write your pallas kernel here")


# speedup vs baseline: 3.5893x; 3.5893x over previous
import functools

import jax
import jax.numpy as jnp
from jax import lax
from jax.experimental import pallas as pl
from jax.experimental.pallas import tpu as pltpu
from jax.experimental.pallas import tpu_sc as plsc

N = 10000
E = 320000
DIN = 128
DH = 64
DF = 128
DOUT = 6
G = 128

NC = 2
NS = 16
NW = NC * NS
CHUNK = 128
STEPS = 79
EPW = CHUNK * STEPS
EP = NW * EPW
NPAD = 10112
RPW = NPAD // NS


def _sc_agg(h, src, dst, zeros):
    mesh = plsc.VectorSubcoreMesh(core_axis_name="c", subcore_axis_name="s")

    @functools.partial(
        pl.kernel,
        mesh=mesh,
        out_type=jax.ShapeDtypeStruct((NC * NPAD, DF), jnp.float32),
        scratch_types=[
            pltpu.VMEM((CHUNK,), jnp.int32),
            pltpu.VMEM((CHUNK,), jnp.int32),
            pltpu.VMEM((CHUNK, DF), jnp.float32),
            pltpu.VMEM_SHARED((NPAD, DF), jnp.float32),
            pltpu.SemaphoreType.DMA,
        ],
    )
    def k(h_hbm, src_hbm, dst_hbm, z_hbm, out_hbm, srcv, dstv, rows, acc, sem):
        c = lax.axis_index("c")
        s = lax.axis_index("s")
        wid = s * NC + c
        pltpu.sync_copy(z_hbm.at[pl.ds(s * RPW, RPW)], acc.at[pl.ds(s * RPW, RPW)])
        plsc.subcore_barrier()

        @pl.loop(0, STEPS)
        def _(t):
            base = wid * EPW + t * CHUNK
            pltpu.sync_copy(src_hbm.at[pl.ds(base, CHUNK)], srcv)
            pltpu.sync_copy(dst_hbm.at[pl.ds(base, CHUNK)], dstv)
            pltpu.async_copy(h_hbm.at[srcv], rows, sem).wait()
            pltpu.sync_copy(rows, acc.at[dstv], add=True)

        plsc.subcore_barrier()
        pltpu.sync_copy(
            acc.at[pl.ds(s * RPW, RPW)],
            out_hbm.at[pl.ds(c * NPAD + s * RPW, RPW)],
        )

    return k(h, src, dst, zeros)


_ROWS = 2000


def _tc_pre(x, W, b):
    def body(x_ref, w_ref, b_ref, o_ref):
        t = (
            jnp.dot(x_ref[...], w_ref[...], preferred_element_type=jnp.float32)
            + b_ref[...]
        )
        o_ref[...] = jnp.concatenate(
            [t, jnp.zeros((_ROWS, DF - DH), jnp.float32)], axis=1
        )

    return pl.pallas_call(
        body,
        grid=(N // _ROWS,),
        in_specs=[
            pl.BlockSpec((_ROWS, DIN), lambda i: (i, 0)),
            pl.BlockSpec((DIN, DH), lambda i: (0, 0)),
            pl.BlockSpec((1, DH), lambda i: (0, 0)),
        ],
        out_specs=pl.BlockSpec((_ROWS, DF), lambda i: (i, 0)),
        out_shape=jax.ShapeDtypeStruct((N, DF), jnp.float32),
    )(x, W, b.reshape(1, DH))


def _tc_mlp(h, a0, a1, Wa, ba, Wb, bb):
    def body(h_ref, a0_ref, a1_ref, wa, bar, wb, bbr, o_ref):
        t = h_ref[...] + a0_ref[...] + a1_ref[...]
        t = t[:, :DH]
        t = jnp.maximum(
            jnp.dot(t, wa[...], preferred_element_type=jnp.float32) + bar[...], 0.0
        )
        t = jnp.dot(t, wb[...], preferred_element_type=jnp.float32) + bbr[...]
        t = jnp.maximum(t, 0.0)
        o_ref[...] = jnp.concatenate(
            [t, jnp.zeros((_ROWS, DF - DH), jnp.float32)], axis=1
        )

    rows_spec = pl.BlockSpec((_ROWS, DF), lambda i: (i, 0))
    w_spec = pl.BlockSpec((DH, DH), lambda i: (0, 0))
    b_spec = pl.BlockSpec((1, DH), lambda i: (0, 0))
    return pl.pallas_call(
        body,
        grid=(N // _ROWS,),
        in_specs=[rows_spec, rows_spec, rows_spec, w_spec, b_spec, w_spec, b_spec],
        out_specs=rows_spec,
        out_shape=jax.ShapeDtypeStruct((N, DF), jnp.float32),
    )(h, a0, a1, Wa, ba.reshape(1, DH), Wb, bb.reshape(1, DH))


def _tc_pool_post(h, batch3, Wp1, bp1, Wp2, bp2):
    nb = N // _ROWS

    def body(h_ref, b_ref, w1, b1r, w2, b2r, o_ref, acc):
        i = pl.program_id(0)

        @pl.when(i == 0)
        def _():
            acc[...] = jnp.zeros_like(acc)

        ids = b_ref[0]
        gi = lax.broadcasted_iota(jnp.int32, (G, _ROWS), 0)
        onehot = (gi == ids).astype(jnp.float32)
        acc[...] += jnp.dot(
            onehot, h_ref[...][:, :DH], preferred_element_type=jnp.float32
        )

        @pl.when(i == nb - 1)
        def _():
            p = acc[...]
            t = jnp.maximum(
                jnp.dot(p, w1[...], preferred_element_type=jnp.float32) + b1r[...],
                0.0,
            )
            o = jnp.dot(t, w2[...], preferred_element_type=jnp.float32) + b2r[...]
            m = jnp.max(o, axis=1, keepdims=True)
            lse = jnp.log(jnp.sum(jnp.exp(o - m), axis=1, keepdims=True)) + m
            o_ref[...] = o - lse

    return pl.pallas_call(
        body,
        grid=(nb,),
        in_specs=[
            pl.BlockSpec((_ROWS, DF), lambda i: (i, 0)),
            pl.BlockSpec((1, 1, _ROWS), lambda i: (i, 0, 0)),
            pl.BlockSpec((DH, DH), lambda i: (0, 0)),
            pl.BlockSpec((1, DH), lambda i: (0, 0)),
            pl.BlockSpec((DH, DOUT), lambda i: (0, 0)),
            pl.BlockSpec((1, DOUT), lambda i: (0, 0)),
        ],
        out_specs=pl.BlockSpec((G, DOUT), lambda i: (0, 0)),
        out_shape=jax.ShapeDtypeStruct((G, DOUT), jnp.float32),
        scratch_shapes=[pltpu.VMEM((G, DH), jnp.float32)],
    )(h, batch3, Wp1, bp1.reshape(1, DH), Wp2, bp2.reshape(1, DOUT))


def kernel(x, edge_index, batch, W_pre, b_pre, W1a, b1a, W1b, b1b, W2a, b2a,
           W2b, b2b, W3a, b3a, W3b, b3b, Wp1, bp1, Wp2, bp2):
    pad = EP - E
    src = jnp.concatenate([edge_index[0], jnp.zeros((pad,), jnp.int32)])
    dst = jnp.concatenate([edge_index[1], jnp.full((pad,), N, jnp.int32)])
    zeros = jnp.zeros((NPAD, DF), jnp.float32)
    batch3 = batch.reshape(N // _ROWS, 1, _ROWS)

    h = _tc_pre(x, W_pre, b_pre)
    for Wa, ba, Wb, bb in ((W1a, b1a, W1b, b1b), (W2a, b2a, W2b, b2b),
                           (W3a, b3a, W3b, b3b)):
        parts = _sc_agg(h, src, dst, zeros)
        a0 = lax.slice(parts, (0, 0), (N, DF))
        a1 = lax.slice(parts, (NPAD, 0), (NPAD + N, DF))
        h = _tc_mlp(h, a0, a1, Wa, ba, Wb, bb)

    return _tc_pool_post(h, batch3, Wp1, bp1, Wp2, bp2)
